# baseline (speedup 1.0000x reference)
import functools

import jax
import jax.numpy as jnp
from jax import lax
from jax.experimental import pallas as pl
from jax.experimental.pallas import tpu as pltpu
from jax.experimental.pallas import tpu_sc as plsc

B = 16384
D = 64
L = 16
NC = 2
NS = 16
NW = NC * NS
BPW = B // NW
CH = 128
CPW = BPW // CH
G = CH // L


def _sc_gather_pairs(idx2d, wpair):
    mesh = plsc.VectorSubcoreMesh(core_axis_name="c", subcore_axis_name="s")

    @functools.partial(
        pl.kernel,
        mesh=mesh,
        out_type=jax.ShapeDtypeStruct((D, B), jnp.float32),
        scratch_types=[
            pltpu.VMEM((CPW, CH), jnp.int32),
            pltpu.VMEM((CPW, CH), jnp.int32),
            pltpu.VMEM((CPW, CH), jnp.int32),
            pltpu.VMEM((CPW, CH, 2 * D), jnp.float32),
            pltpu.VMEM((D, BPW), jnp.float32),
            pltpu.SemaphoreType.DMA,
        ],
        compiler_params=pltpu.CompilerParams(
            use_tc_tiling_on_sc=True, needs_layout_passes=False
        ),
    )
    def k(idx_hbm, w_hbm, out_hbm, idx_v, pidx_v, h64_v, rows_v, col_v, sem):
        wid = lax.axis_index("s") * NC + lax.axis_index("c")
        base = wid * BPW
        pltpu.sync_copy(idx_hbm.at[pl.ds(wid * CPW, CPW)], idx_v)

        for j in range(CPW):
            for g in range(G):
                sl = pl.ds(g * L, L)
                v = idx_v[j, sl]
                pidx_v[j, sl] = lax.shift_right_logical(v, 1)
                h64_v[j, sl] = lax.shift_left(v & 1, 6)

        descs = []
        for j in range(CPW):
            descs.append(
                pltpu.async_copy(
                    w_hbm.at[pidx_v.at[j]], rows_v.at[j], sem
                )
            )
        for d in descs:
            d.wait()

        lanes = lax.iota(jnp.int32, L)
        for j in range(CPW):
            jvec = jnp.full((L,), j, jnp.int32)

            def extract(c, carry, j=j, jvec=jvec):
                for g in range(G):
                    sl = pl.ds(g * L, L)
                    cols = h64_v[j, sl] + c
                    vals = plsc.load_gather(rows_v, [jvec, lanes + (g * L), cols])
                    col_v[c, pl.ds(j * CH + g * L, L)] = vals
                return carry

            lax.fori_loop(0, D, extract, 0)

        pltpu.sync_copy(col_v, out_hbm.at[:, pl.ds(base, BPW)])

    return k(idx2d, wpair)


def kernel(idx, weight):
    idx2d = idx.astype(jnp.int32).reshape(B // CH, CH)
    wpair = weight.reshape(500000, 2 * D)
    out_t = _sc_gather_pairs(idx2d, wpair)
    return out_t.T
